# baseline (speedup 1.0000x reference)
import functools

import jax
import jax.numpy as jnp
from jax import lax
from jax.experimental import pallas as pl
from jax.experimental.pallas import tpu as pltpu

_EPS = 1e-8


def _pool_kernel(x_ref, out_ref, *, nblk, inv_s):
    i = pl.program_id(0)

    @pl.when(i == 0)
    def _():
        out_ref[...] = jnp.zeros_like(out_ref)

    out_ref[...] += jnp.sum(x_ref[...], axis=1)

    @pl.when(i == nblk - 1)
    def _():
        out_ref[...] = out_ref[...] * inv_s


def _fused_kernel(q_ref, x_ref, e_ref, sl_ref,
                  wctx_ref, bctx_ref, w1_ref, b1_ref, w2_ref, b2_ref,
                  raw_ref,
                  pred_ref, ps_ref, best_ref, scale_ref,
                  mx_ref, idx_ref, carry_ref, row_ref, dsem,
                  *, nblk, blk_per_seq, batch, t):
    i = pl.program_id(0)

    @pl.when(i == 0)
    def _():
        mx_ref[...] = jnp.full_like(mx_ref, -jnp.inf)
        idx_ref[...] = jnp.zeros_like(idx_ref)

    cur = x_ref[...]
    prev_row = jnp.where(i % blk_per_seq == 0, 0.0, carry_ref[...])
    shifted = jnp.concatenate([prev_row, cur[:t - 1, :]], axis=0)
    carry_ref[...] = cur[t - 1:t, :]

    bf = jnp.bfloat16
    ctx = jnp.dot(shifted.astype(bf), wctx_ref[...],
                  preferred_element_type=jnp.float32) + bctx_ref[...]
    h = jnp.dot(ctx.astype(bf), w1_ref[...],
                preferred_element_type=jnp.float32) + b1_ref[...]
    h = 0.5 * h * (1.0 + lax.erf(h * 0.7071067811865476))
    pred = jnp.dot(h.astype(bf), w2_ref[...],
                   preferred_element_type=jnp.float32) + b2_ref[...]
    pred_ref[...] = pred

    diff = cur - pred
    mse = jnp.mean(diff * diff, axis=1, keepdims=True)
    ps_ref[...] = jnp.sqrt(mse)

    e = e_ref[...]
    k = e.shape[0]
    ebf = e.astype(jnp.bfloat16)
    esq = ebf * ebf
    ones = jnp.ones((1, e.shape[1]), jnp.bfloat16)
    norms2 = lax.dot_general(ones, esq,
                             (((1,), (1,)), ((), ())),
                             preferred_element_type=jnp.float32)
    inv_nb = lax.rsqrt(jnp.maximum(norms2, _EPS * _EPS))
    dots = lax.dot_general(q_ref[...], e,
                           (((1,), (1,)), ((), ())),
                           preferred_element_type=jnp.float32)
    active = sl_ref[0] > 0.0
    ids = lax.broadcasted_iota(jnp.int32, (1, k), 1)

    for b in range(batch):
        qb = q_ref[pl.ds(b, 1), :]
        na = jnp.maximum(jnp.sqrt(jnp.sum(qb * qb)), _EPS)
        simb = dots[b:b + 1, :] * (inv_nb / na)
        simb = jnp.where(active, simb, -jnp.inf)
        mxv = jnp.max(simb, axis=1, keepdims=True)
        locv = jnp.min(jnp.where(simb == mxv, ids, k),
                       axis=1, keepdims=True) + i * k
        lanes = mx_ref.shape[1]
        bcmx = jnp.zeros((1, lanes), jnp.float32) + mxv
        bcidx = jnp.zeros((1, lanes), jnp.int32) + locv
        run_mx = mx_ref[pl.ds(b, 1), :]
        upd = bcmx > run_mx
        mx_ref[pl.ds(b, 1), :] = jnp.where(upd, bcmx, run_mx)
        idx_ref[pl.ds(b, 1), :] = jnp.where(upd, bcidx,
                                            idx_ref[pl.ds(b, 1), :])

    @pl.when(i == nblk - 1)
    def _():
        for b in range(batch):
            gidx = idx_ref[b, 0]
            cp = pltpu.make_async_copy(
                raw_ref.at[pl.ds(gidx, 1), :],
                row_ref.at[pl.ds(b, 1), :], dsem)
            cp.start()
            cp.wait()
            val = jnp.where(mx_ref[b, 0] > 0.3, 2.0, 0.0)
            scale_ref[pl.ds(b, 1), :, :] = (
                jnp.zeros((1, 1, scale_ref.shape[-1]), jnp.float32) + val)
        best_ref[...] = row_ref[...].reshape(best_ref.shape)


def _combine_kernel(x_ref, ps_ref, best_ref, scale_ref, sur_ref):
    cur = x_ref[0]
    bb = best_ref[0]
    bn = jnp.maximum(jnp.sqrt(jnp.sum(bb * bb)), _EPS)
    dot = jnp.sum(cur * bb, axis=1, keepdims=True)
    xn = jnp.maximum(jnp.sqrt(jnp.sum(cur * cur, axis=1, keepdims=True)),
                     _EPS)
    cosv = dot / (xn * bn)
    contr = scale_ref[0, 0, 0] * (1.0 - cosv)
    sur_ref[0, 0] = jnp.maximum(ps_ref[0, 0], contr)


@jax.jit
def kernel(x, W_ctx, b_ctx, W1, b1, W2, b2, raw_embeddings, surprise_level):
    B, S, H = x.shape
    SLOTS = raw_embeddings.shape[0]

    TP = 512
    npool = S // TP
    pooled = pl.pallas_call(
        functools.partial(_pool_kernel, nblk=npool, inv_s=1.0 / S),
        grid=(npool,),
        in_specs=[pl.BlockSpec((B, TP, H), lambda i: (0, i, 0))],
        out_specs=pl.BlockSpec((B, H), lambda i: (0, 0)),
        out_shape=jax.ShapeDtypeStruct((B, H), jnp.float32),
        compiler_params=pltpu.CompilerParams(
            dimension_semantics=("arbitrary",)),
    )(x)

    NBLK = 32
    T = (B * S) // NBLK
    K = SLOTS // NBLK
    blk_per_seq = S // T
    xf = x.reshape(B * S, H)
    sl3 = surprise_level.reshape(NBLK, 1, K)
    wctx_t = W_ctx.T.astype(jnp.bfloat16)
    w1_t = W1.T.astype(jnp.bfloat16)
    w2_t = W2.T.astype(jnp.bfloat16)
    bctx2, b12, b22 = (b_ctx.reshape(1, H), b1.reshape(1, H),
                       b2.reshape(1, H))
    predf, psf, best, scale = pl.pallas_call(
        functools.partial(_fused_kernel, nblk=NBLK, blk_per_seq=blk_per_seq,
                          batch=B, t=T),
        grid=(NBLK,),
        in_specs=[
            pl.BlockSpec((B, H), lambda i: (0, 0)),
            pl.BlockSpec((T, H), lambda i: (i, 0)),
            pl.BlockSpec((K, H), lambda i: (i, 0)),
            pl.BlockSpec((1, 1, K), lambda i: (i, 0, 0)),
            pl.BlockSpec((H, H), lambda i: (0, 0)),
            pl.BlockSpec((1, H), lambda i: (0, 0)),
            pl.BlockSpec((H, H), lambda i: (0, 0)),
            pl.BlockSpec((1, H), lambda i: (0, 0)),
            pl.BlockSpec((H, H), lambda i: (0, 0)),
            pl.BlockSpec((1, H), lambda i: (0, 0)),
            pl.BlockSpec(memory_space=pltpu.MemorySpace.HBM),
        ],
        out_specs=[
            pl.BlockSpec((T, H), lambda i: (i, 0)),
            pl.BlockSpec((T, 1), lambda i: (i, 0)),
            pl.BlockSpec((B, 1, H), lambda i: (0, 0, 0)),
            pl.BlockSpec((B, 1, 128), lambda i: (0, 0, 0)),
        ],
        out_shape=[
            jax.ShapeDtypeStruct((B * S, H), jnp.float32),
            jax.ShapeDtypeStruct((B * S, 1), jnp.float32),
            jax.ShapeDtypeStruct((B, 1, H), jnp.float32),
            jax.ShapeDtypeStruct((B, 1, 128), jnp.float32),
        ],
        scratch_shapes=[pltpu.VMEM((B, 128), jnp.float32),
                        pltpu.VMEM((B, 128), jnp.int32),
                        pltpu.VMEM((1, H), jnp.float32),
                        pltpu.VMEM((B, H), jnp.float32),
                        pltpu.SemaphoreType.DMA],
        compiler_params=pltpu.CompilerParams(
            dimension_semantics=("arbitrary",)),
    )(pooled, xf, raw_embeddings, sl3,
      wctx_t, bctx2, w1_t, b12, w2_t, b22, raw_embeddings)

    TC = 512
    nc = S // TC
    ps4 = psf.reshape(B, nc, TC, 1)
    pred, sur4 = (predf.reshape(B, S, H), pl.pallas_call(
        _combine_kernel,
        grid=(B, nc),
        in_specs=[
            pl.BlockSpec((1, TC, H), lambda b, i: (b, i, 0)),
            pl.BlockSpec((1, 1, TC, 1), lambda b, i: (b, i, 0, 0)),
            pl.BlockSpec((1, 1, H), lambda b, i: (b, 0, 0)),
            pl.BlockSpec((1, 1, 128), lambda b, i: (b, 0, 0)),
        ],
        out_specs=pl.BlockSpec((1, 1, TC, 1), lambda b, i: (b, i, 0, 0)),
        out_shape=jax.ShapeDtypeStruct((B, nc, TC, 1), jnp.float32),
        compiler_params=pltpu.CompilerParams(
            dimension_semantics=("arbitrary", "arbitrary")),
    )(x, ps4, best, scale))

    surprise = sur4.reshape(B, S)
    return (surprise, pred)


# speedup vs baseline: 1.0640x; 1.0640x over previous
import functools

import jax
import jax.numpy as jnp
from jax import lax
from jax.experimental import pallas as pl
from jax.experimental.pallas import tpu as pltpu

_EPS = 1e-8


def _pool_kernel(x_ref, out_ref, *, nblk, inv_s):
    i = pl.program_id(0)

    @pl.when(i == 0)
    def _():
        out_ref[...] = jnp.zeros_like(out_ref)

    out_ref[...] += jnp.sum(x_ref[...], axis=1)

    @pl.when(i == nblk - 1)
    def _():
        out_ref[...] = out_ref[...] * inv_s


def _fused_kernel(q_ref, x_ref, e_ref, sl_ref,
                  wctx_ref, bctx_ref, w1_ref, b1_ref, w2_ref, b2_ref,
                  raw_ref,
                  pred_ref, ps_ref, best_ref, scale_ref,
                  mx_ref, idx_ref, carry_ref, row_ref, dsem,
                  *, nblk, blk_per_seq, batch, t):
    i = pl.program_id(0)

    @pl.when(i == 0)
    def _():
        mx_ref[...] = jnp.full_like(mx_ref, -jnp.inf)
        idx_ref[...] = jnp.zeros_like(idx_ref)

    e = e_ref[...]
    k = e.shape[0]
    ebf = e.astype(jnp.bfloat16)
    esq = ebf * ebf
    ones = jnp.ones((1, e.shape[1]), jnp.bfloat16)
    norms2 = lax.dot_general(ones, esq,
                             (((1,), (1,)), ((), ())),
                             preferred_element_type=jnp.float32)
    inv_nb = lax.rsqrt(jnp.maximum(norms2, _EPS * _EPS))
    dots = lax.dot_general(q_ref[...], e,
                           (((1,), (1,)), ((), ())),
                           preferred_element_type=jnp.float32)
    active = sl_ref[0] > 0.0
    ids = lax.broadcasted_iota(jnp.int32, (1, k), 1)

    for b in range(batch):
        qb = q_ref[pl.ds(b, 1), :]
        na = jnp.maximum(jnp.sqrt(jnp.sum(qb * qb)), _EPS)
        simb = dots[b:b + 1, :] * (inv_nb / na)
        simb = jnp.where(active, simb, -jnp.inf)
        mxv = jnp.max(simb, axis=1, keepdims=True)
        locv = jnp.min(jnp.where(simb == mxv, ids, k),
                       axis=1, keepdims=True) + i * k
        lanes = mx_ref.shape[1]
        bcmx = jnp.zeros((1, lanes), jnp.float32) + mxv
        bcidx = jnp.zeros((1, lanes), jnp.int32) + locv
        run_mx = mx_ref[pl.ds(b, 1), :]
        upd = bcmx > run_mx
        mx_ref[pl.ds(b, 1), :] = jnp.where(upd, bcmx, run_mx)
        idx_ref[pl.ds(b, 1), :] = jnp.where(upd, bcidx,
                                            idx_ref[pl.ds(b, 1), :])

    cur = x_ref[...]
    prev_row = jnp.where(i % blk_per_seq == 0, 0.0, carry_ref[...])
    shifted = jnp.concatenate([prev_row, cur[:t - 1, :]], axis=0)
    carry_ref[...] = cur[t - 1:t, :]

    bf = jnp.bfloat16
    ctx = jnp.dot(shifted.astype(bf), wctx_ref[...],
                  preferred_element_type=jnp.float32) + bctx_ref[...]
    h = jnp.dot(ctx.astype(bf), w1_ref[...],
                preferred_element_type=jnp.float32) + b1_ref[...]
    h = 0.5 * h * (1.0 + lax.erf(h * 0.7071067811865476))
    pred = jnp.dot(h.astype(bf), w2_ref[...],
                   preferred_element_type=jnp.float32) + b2_ref[...]
    pred_ref[...] = pred

    diff = cur - pred
    mse = jnp.mean(diff * diff, axis=1, keepdims=True)
    ps_ref[...] = jnp.sqrt(mse)

    @pl.when(i == nblk - 1)
    def _():
        for b in range(batch):
            gidx = idx_ref[b, 0]
            cp = pltpu.make_async_copy(
                raw_ref.at[pl.ds(gidx, 1), :],
                row_ref.at[pl.ds(b, 1), :], dsem)
            cp.start()
            cp.wait()
            val = jnp.where(mx_ref[b, 0] > 0.3, 2.0, 0.0)
            scale_ref[pl.ds(b, 1), :, :] = (
                jnp.zeros((1, 1, scale_ref.shape[-1]), jnp.float32) + val)
        best_ref[...] = row_ref[...].reshape(best_ref.shape)


def _combine_kernel(x_ref, ps_ref, best_ref, scale_ref, sur_ref):
    cur = x_ref[0]
    bb = best_ref[0]
    bn = jnp.maximum(jnp.sqrt(jnp.sum(bb * bb)), _EPS)
    dot = jnp.sum(cur * bb, axis=1, keepdims=True)
    xn = jnp.maximum(jnp.sqrt(jnp.sum(cur * cur, axis=1, keepdims=True)),
                     _EPS)
    cosv = dot / (xn * bn)
    contr = scale_ref[0, 0, 0] * (1.0 - cosv)
    sur_ref[0, 0] = jnp.maximum(ps_ref[0, 0], contr)


@jax.jit
def kernel(x, W_ctx, b_ctx, W1, b1, W2, b2, raw_embeddings, surprise_level):
    B, S, H = x.shape
    SLOTS = raw_embeddings.shape[0]

    TP = 512
    npool = S // TP
    pooled = pl.pallas_call(
        functools.partial(_pool_kernel, nblk=npool, inv_s=1.0 / S),
        grid=(npool,),
        in_specs=[pl.BlockSpec((B, TP, H), lambda i: (0, i, 0))],
        out_specs=pl.BlockSpec((B, H), lambda i: (0, 0)),
        out_shape=jax.ShapeDtypeStruct((B, H), jnp.float32),
        compiler_params=pltpu.CompilerParams(
            dimension_semantics=("arbitrary",)),
    )(x)

    NBLK = 16
    T = (B * S) // NBLK
    K = SLOTS // NBLK
    blk_per_seq = S // T
    xf = x.reshape(B * S, H)
    sl3 = surprise_level.reshape(NBLK, 1, K)
    wctx_t = W_ctx.T.astype(jnp.bfloat16)
    w1_t = W1.T.astype(jnp.bfloat16)
    w2_t = W2.T.astype(jnp.bfloat16)
    bctx2, b12, b22 = (b_ctx.reshape(1, H), b1.reshape(1, H),
                       b2.reshape(1, H))
    predf, psf, best, scale = pl.pallas_call(
        functools.partial(_fused_kernel, nblk=NBLK, blk_per_seq=blk_per_seq,
                          batch=B, t=T),
        grid=(NBLK,),
        in_specs=[
            pl.BlockSpec((B, H), lambda i: (0, 0)),
            pl.BlockSpec((T, H), lambda i: (i, 0)),
            pl.BlockSpec((K, H), lambda i: (i, 0)),
            pl.BlockSpec((1, 1, K), lambda i: (i, 0, 0)),
            pl.BlockSpec((H, H), lambda i: (0, 0)),
            pl.BlockSpec((1, H), lambda i: (0, 0)),
            pl.BlockSpec((H, H), lambda i: (0, 0)),
            pl.BlockSpec((1, H), lambda i: (0, 0)),
            pl.BlockSpec((H, H), lambda i: (0, 0)),
            pl.BlockSpec((1, H), lambda i: (0, 0)),
            pl.BlockSpec(memory_space=pltpu.MemorySpace.HBM),
        ],
        out_specs=[
            pl.BlockSpec((T, H), lambda i: (i, 0)),
            pl.BlockSpec((T, 1), lambda i: (i, 0)),
            pl.BlockSpec((B, 1, H), lambda i: (0, 0, 0)),
            pl.BlockSpec((B, 1, 128), lambda i: (0, 0, 0)),
        ],
        out_shape=[
            jax.ShapeDtypeStruct((B * S, H), jnp.float32),
            jax.ShapeDtypeStruct((B * S, 1), jnp.float32),
            jax.ShapeDtypeStruct((B, 1, H), jnp.float32),
            jax.ShapeDtypeStruct((B, 1, 128), jnp.float32),
        ],
        scratch_shapes=[pltpu.VMEM((B, 128), jnp.float32),
                        pltpu.VMEM((B, 128), jnp.int32),
                        pltpu.VMEM((1, H), jnp.float32),
                        pltpu.VMEM((B, H), jnp.float32),
                        pltpu.SemaphoreType.DMA],
        compiler_params=pltpu.CompilerParams(
            dimension_semantics=("arbitrary",)),
    )(pooled, xf, raw_embeddings, sl3,
      wctx_t, bctx2, w1_t, b12, w2_t, b22, raw_embeddings)

    TC = 512
    nc = S // TC
    ps4 = psf.reshape(B, nc, TC, 1)
    pred, sur4 = (predf.reshape(B, S, H), pl.pallas_call(
        _combine_kernel,
        grid=(B, nc),
        in_specs=[
            pl.BlockSpec((1, TC, H), lambda b, i: (b, i, 0)),
            pl.BlockSpec((1, 1, TC, 1), lambda b, i: (b, i, 0, 0)),
            pl.BlockSpec((1, 1, H), lambda b, i: (b, 0, 0)),
            pl.BlockSpec((1, 1, 128), lambda b, i: (b, 0, 0)),
        ],
        out_specs=pl.BlockSpec((1, 1, TC, 1), lambda b, i: (b, i, 0, 0)),
        out_shape=jax.ShapeDtypeStruct((B, nc, TC, 1), jnp.float32),
        compiler_params=pltpu.CompilerParams(
            dimension_semantics=("arbitrary", "arbitrary")),
    )(x, ps4, best, scale))

    surprise = sur4.reshape(B, S)
    return (surprise, pred)
